# Initial kernel scaffold; baseline (speedup 1.0000x reference)
import jax
import jax.numpy as jnp
from jax.experimental import pallas as pl


def kernel(x, edge_index, batch, W_gcn, b_gcn, W_lin, b_lin):
    raise NotImplementedError("write your pallas kernel here")


# speedup vs baseline: 37.4708x; 37.4708x over previous
import functools

import jax
import jax.numpy as jnp
from jax import lax
from jax.experimental import pallas as pl
from jax.experimental.pallas import tpu as pltpu
from jax.experimental.pallas import tpu_sc as plsc

N = 10000
NPAD = 10240
E = 320000
IN_F = 128
HID_F = 64
OUT_F = 64
B = 64

NC = 2
NS = 16
NW = NC * NS
CHUNK = 80
NCHUNK = E // (NW * CHUNK)
NCHUNK_PAD = 128
ROWS_TILE = NPAD // NS

_sc_mesh = plsc.VectorSubcoreMesh(
    core_axis_name="c", subcore_axis_name="s", num_cores=NC, num_subcores=NS)
_sc_params = pltpu.CompilerParams(use_tc_tiling_on_sc=False)


@functools.partial(
    pl.kernel,
    out_type=jax.ShapeDtypeStruct((NC * NPAD,), jnp.float32),
    mesh=_sc_mesh,
    compiler_params=_sc_params,
    scratch_types=[
        pltpu.VMEM((NCHUNK_PAD, CHUNK), jnp.int32),
        pltpu.VMEM((CHUNK,), jnp.float32),
        pltpu.VMEM((ROWS_TILE,), jnp.float32),
        pltpu.VMEM_SHARED((NPAD,), jnp.float32),
        pltpu.SemaphoreType.DMA,
    ],
)
def _deg_sc(dst_hbm, zvec_hbm, degp_hbm, didx, ones_v, bounce, deg_sh, sem):
    c = lax.axis_index("c")
    s = lax.axis_index("s")
    wid = c * NS + s
    for i in range(CHUNK // 16):
        ones_v[pl.ds(i * 16, 16)] = jnp.ones((16,), jnp.float32)
    pltpu.sync_copy(zvec_hbm, deg_sh.at[pl.ds(s * ROWS_TILE, ROWS_TILE)])
    pltpu.sync_copy(dst_hbm.at[pl.ds(wid * NCHUNK_PAD, NCHUNK_PAD), :], didx)
    plsc.subcore_barrier()
    def body(i, carry):
        for k in range(5):
            pltpu.async_copy(ones_v, deg_sh.at[didx.at[i * 5 + k]], sem,
                             add=True)
        for k in range(5):
            pltpu.make_async_copy(ones_v, deg_sh.at[didx.at[0]], sem).wait()
        return carry
    lax.fori_loop(0, NCHUNK // 5, body, 0)
    plsc.subcore_barrier()
    pltpu.sync_copy(deg_sh.at[pl.ds(s * ROWS_TILE, ROWS_TILE)], bounce)
    pltpu.sync_copy(bounce,
                    degp_hbm.at[pl.ds(c * NPAD + s * ROWS_TILE, ROWS_TILE)])


@functools.partial(
    pl.kernel,
    out_type=jax.ShapeDtypeStruct((NC, NPAD, HID_F), jnp.float32),
    mesh=_sc_mesh,
    compiler_params=_sc_params,
    scratch_types=[
        pltpu.VMEM((NCHUNK_PAD, CHUNK), jnp.int32),
        pltpu.VMEM((NCHUNK_PAD, CHUNK), jnp.int32),
        pltpu.VMEM((2, CHUNK, HID_F), jnp.float32),
        pltpu.VMEM((CHUNK, HID_F), jnp.float32),
        pltpu.VMEM_SHARED((NPAD, HID_F), jnp.float32),
        pltpu.SemaphoreType.DMA,
        pltpu.SemaphoreType.DMA,
    ],
)
def _msg_sc(src_hbm, dst_hbm, g_hbm, zrows_hbm, accp_hbm,
            sidx, didx, rows, bounce, acc_sh, gsem, ssem):
    c = lax.axis_index("c")
    s = lax.axis_index("s")
    wid = c * NS + s

    pltpu.sync_copy(zrows_hbm, acc_sh.at[pl.ds(s * ROWS_TILE, ROWS_TILE), :])
    pltpu.sync_copy(src_hbm.at[pl.ds(wid * NCHUNK_PAD, NCHUNK_PAD), :], sidx)
    pltpu.sync_copy(dst_hbm.at[pl.ds(wid * NCHUNK_PAD, NCHUNK_PAD), :], didx)
    plsc.subcore_barrier()

    def fire_gather(j, b):
        pltpu.async_copy(g_hbm.at[sidx.at[j]], rows.at[b], gsem)

    def wait_gather(b):
        pltpu.make_async_copy(g_hbm.at[sidx.at[0]], rows.at[b], gsem).wait()

    def fire_scatter(j, b):
        pltpu.async_copy(rows.at[b], acc_sh.at[didx.at[j]], ssem, add=True)

    def wait_scatter(b):
        pltpu.make_async_copy(rows.at[b], acc_sh.at[didx.at[0]], ssem).wait()

    fire_gather(0, 0)
    wait_gather(0)
    fire_scatter(0, 0)
    fire_gather(1, 1)

    def body(i, carry):
        j = 1 + 2 * i
        wait_gather(1)
        fire_scatter(j, 1)
        wait_scatter(0)
        fire_gather(j + 1, 0)
        wait_gather(0)
        fire_scatter(j + 1, 0)
        wait_scatter(1)
        fire_gather(j + 2, 1)
        return carry
    lax.fori_loop(0, (NCHUNK - 3) // 2, body, 0)

    j = NCHUNK - 2
    wait_gather(1)
    fire_scatter(j, 1)
    wait_scatter(0)
    fire_gather(j + 1, 0)
    wait_gather(0)
    fire_scatter(j + 1, 0)
    wait_scatter(1)
    wait_scatter(0)

    plsc.subcore_barrier()
    for k in range(ROWS_TILE // CHUNK):
        r0 = s * ROWS_TILE + k * CHUNK
        pltpu.sync_copy(acc_sh.at[pl.ds(r0, CHUNK), :], bounce)
        pltpu.sync_copy(bounce, accp_hbm.at[c, pl.ds(r0, CHUNK), :])


def _mm_body(x_ref, w_ref, degp_ref, g_ref, dinv_ref):
    d = degp_ref[0] + degp_ref[1] + 1.0
    dinv = lax.rsqrt(d)
    h = jnp.dot(x_ref[...], w_ref[...], preferred_element_type=jnp.float32)
    g_ref[...] = h * dinv
    dinv_ref[...] = dinv


_mm_tc = pl.pallas_call(
    _mm_body,
    out_shape=(jax.ShapeDtypeStruct((NPAD, HID_F), jnp.float32),
               jax.ShapeDtypeStruct((NPAD, 1), jnp.float32)),
)


def _fin_body(accp_ref, g_ref, dinv_ref, bg_ref, batch_ref, wl_ref, bl_ref,
              out_ref):
    acc = accp_ref[0] + accp_ref[1] + g_ref[...]
    act = jnp.maximum(acc * dinv_ref[...] + bg_ref[...], 0.0)
    ids = lax.broadcasted_iota(jnp.int32, (B, NPAD), 0)
    onehot = jnp.where(ids == batch_ref[...], 1.0, 0.0)
    seg = jnp.dot(onehot, act, preferred_element_type=jnp.float32)
    cnt = jnp.sum(onehot, axis=1, keepdims=True)
    emb = seg / jnp.maximum(cnt, 1.0)
    z = jnp.dot(emb, wl_ref[...], preferred_element_type=jnp.float32)
    out_ref[...] = jnp.tanh(z + bl_ref[...])


_fin_tc = pl.pallas_call(
    _fin_body,
    out_shape=jax.ShapeDtypeStruct((B, OUT_F), jnp.float32),
)


def _pad_chunks(v):
    v3 = v.reshape(NW, NCHUNK, CHUNK)
    v3 = jnp.pad(v3, ((0, 0), (0, NCHUNK_PAD - NCHUNK), (0, 0)))
    return v3.reshape(NW * NCHUNK_PAD, CHUNK)


def kernel(x, edge_index, batch, W_gcn, b_gcn, W_lin, b_lin):
    src = _pad_chunks(edge_index[0])
    dst = _pad_chunks(edge_index[1])
    zvec = jnp.zeros((ROWS_TILE,), jnp.float32)
    zrows = jnp.zeros((ROWS_TILE, HID_F), jnp.float32)

    degp = _deg_sc(dst, zvec).reshape(NC, NPAD)
    x_pad = jnp.pad(x, ((0, NPAD - N), (0, 0)))
    g, dinv = _mm_tc(x_pad, W_gcn, degp.reshape(NC, NPAD, 1))
    accp = _msg_sc(src, dst, g, zrows)
    batch_pad = jnp.pad(batch, (0, NPAD - N),
                        constant_values=B).reshape(1, NPAD)
    return _fin_tc(accp, g, dinv, b_gcn.reshape(1, HID_F), batch_pad,
                   W_lin, b_lin.reshape(1, OUT_F))
